# baseline (speedup 1.0000x reference)
import functools

import jax
import jax.numpy as jnp
from jax import lax
from jax.experimental import pallas as pl
from jax.experimental.pallas import tpu as pltpu
from jax.experimental.pallas import tpu_sc as plsc

_D = 32
_E = 200000
_NW = 32
_CHUNK = 128
_CH = 50
_EPAD = _NW * _CH * _CHUNK
_BT = 2048
_BR = _BT // 4
_GB = _EPAD // _BT


def _sc_gather(node_emb, head_idx, tail_idx):
    mesh = plsc.VectorSubcoreMesh(core_axis_name="c", subcore_axis_name="s")

    @functools.partial(
        pl.kernel,
        mesh=mesh,
        out_type=jax.ShapeDtypeStruct((_EPAD // 4, 128), jnp.bfloat16),
        scratch_types=[
            pltpu.VMEM((_CH, _CHUNK), jnp.int32),
            pltpu.VMEM((_CH, _CHUNK), jnp.int32),
            [pltpu.VMEM((_CHUNK, _D), jnp.bfloat16) for _ in range(2)],
            [pltpu.VMEM((_CHUNK, _D), jnp.bfloat16) for _ in range(2)],
            [pltpu.VMEM((_CHUNK, _D), jnp.bfloat16) for _ in range(2)],
            [pltpu.SemaphoreType.DMA for _ in range(2)],
            [pltpu.SemaphoreType.DMA for _ in range(2)],
        ],
        compiler_params=pltpu.CompilerParams(use_tc_tiling_on_sc=False),
    )
    def gather_kernel(node_hbm, hidx_hbm, tidx_hbm, dout_hbm,
                      hidx_v, tidx_v, hbuf, tbuf, dbuf, sem_g, sem_w):
        wid = lax.axis_index("s") * 2 + lax.axis_index("c")
        pltpu.sync_copy(hidx_hbm.at[wid], hidx_v)
        pltpu.sync_copy(tidx_hbm.at[wid], tidx_v)

        def start_gather(c, b):
            pltpu.make_async_copy(node_hbm.at[hidx_v.at[c]], hbuf[b],
                                  sem_g[b]).start()
            pltpu.make_async_copy(node_hbm.at[tidx_v.at[c]], tbuf[b],
                                  sem_g[b]).start()

        def wait_gather(b):
            pltpu.make_async_copy(node_hbm.at[hidx_v.at[0]], hbuf[b],
                                  sem_g[b]).wait()
            pltpu.make_async_copy(node_hbm.at[tidx_v.at[0]], tbuf[b],
                                  sem_g[b]).wait()

        def wait_write(b):
            pltpu.make_async_copy(
                dbuf[b], dout_hbm.at[pl.ds(0, _CHUNK), pl.ds(0, _D)],
                sem_w[b]).wait()

        def start_write(c, b):
            t = wid * _CH + c
            r0 = (t >> 4) * 512 + (t & 3) * _CHUNK
            lane = ((t >> 2) & 3) * _D
            pltpu.make_async_copy(
                dbuf[b], dout_hbm.at[pl.ds(r0, _CHUNK), pl.ds(lane, _D)],
                sem_w[b]).start()

        start_gather(0, 0)

        def pair(g, carry):
            c0 = g * 2

            @pl.when(g > 0)
            def _w0():
                wait_write(0)

            wait_gather(0)
            start_gather(c0 + 1, 1)

            def sub_row0(r, carry2):
                r8 = r * 8
                for q in range(8):
                    dbuf[0][r8 + q, :] = hbuf[0][r8 + q, :] - tbuf[0][r8 + q, :]
                return carry2

            lax.fori_loop(0, _CHUNK // 8, sub_row0, 0)
            start_write(c0, 0)

            @pl.when(g > 0)
            def _w1():
                wait_write(1)

            wait_gather(1)

            @pl.when(g < _CH // 2 - 1)
            def _ng():
                start_gather(c0 + 2, 0)

            def sub_row1(r, carry2):
                r8 = r * 8
                for q in range(8):
                    dbuf[1][r8 + q, :] = hbuf[1][r8 + q, :] - tbuf[1][r8 + q, :]
                return carry2

            lax.fori_loop(0, _CHUNK // 8, sub_row1, 0)
            start_write(c0 + 1, 1)
            return carry

        lax.fori_loop(0, _CH // 2, pair, 0)
        wait_write(0)
        wait_write(1)

    return gather_kernel(node_emb, head_idx, tail_idx)


def _tc_loss(diff2d, ridx3, p_stacked, r_emb_w):
    def body(d_ref, r_ref, p_ref, e_ref, o_ref):
        i = pl.program_id(0)
        blk_t = d_ref[...].T

        growt = lax.shift_right_logical(
            lax.broadcasted_iota(jnp.int32, (_D * _D, _BR), 0), 5)
        rrow = lax.broadcasted_iota(jnp.int32, (64, _BR), 0)
        kcol = lax.broadcasted_iota(jnp.int32, (1, _BR), 1)
        p_t = p_ref[...].astype(jnp.bfloat16)
        e_t = e_ref[...]

        part = jnp.zeros((), jnp.float32)
        for j in range(4):
            d_t = blk_t[_D * j:_D * (j + 1), :]
            rj = r_ref[0, j, :]
            g = lax.shift_right_logical(rj, 1)

            tile_t = jnp.concatenate([d_t] * _D, axis=0)
            sel = (growt == g[None, :])
            x_t = jnp.where(sel, tile_t, jnp.bfloat16(0))
            out_t = jnp.dot(p_t, x_t,
                            preferred_element_type=jnp.float32)

            onehot_t = (rrow == rj[None, :]).astype(jnp.float32)
            re_t = jnp.dot(e_t, onehot_t,
                           preferred_element_type=jnp.float32)

            s = out_t + re_t
            e_glob = i * _BT + j * _BR + kcol
            s = jnp.where(e_glob < _E, s, 0.0)
            part = part + jnp.sum(s * s)

        @pl.when(i == 0)
        def _init():
            o_ref[...] = jnp.zeros((1, 1), jnp.float32)

        o_ref[...] = o_ref[...] + part

        @pl.when(i == _GB - 1)
        def _final():
            o_ref[...] = o_ref[...] * (1.0 / (_E * _D))

    return pl.pallas_call(
        body,
        grid=(_GB,),
        in_specs=[
            pl.BlockSpec((_BR, 128), lambda i: (i, 0)),
            pl.BlockSpec((1, 8, _BR), lambda i: (i, 0, 0)),
            pl.BlockSpec((_D, _D * _D), lambda i: (0, 0)),
            pl.BlockSpec((_D, 64), lambda i: (0, 0)),
        ],
        out_specs=pl.BlockSpec((1, 1), lambda i: (0, 0)),
        out_shape=jax.ShapeDtypeStruct((1, 1), jnp.float32),
    )(diff2d, ridx3, p_stacked, r_emb_w)


def kernel(node_emb, r_emb_w, r_proj_w, edge_index_t, edge_attr):
    pad = _EPAD - _E
    head_idx = jnp.concatenate(
        [edge_index_t[:, 0], jnp.zeros((pad,), jnp.int32)]).astype(jnp.int32)
    tail_idx = jnp.concatenate(
        [edge_index_t[:, 1], jnp.zeros((pad,), jnp.int32)]).astype(jnp.int32)
    head_idx = head_idx.reshape(_NW, _CH, _CHUNK)
    tail_idx = tail_idx.reshape(_NW, _CH, _CHUNK)

    diff_rows = _sc_gather(node_emb.astype(jnp.bfloat16), head_idx, tail_idx)

    ridx = jnp.concatenate(
        [edge_attr[:, 0], jnp.zeros((pad,), jnp.int32)]).astype(jnp.int32)
    ridx3 = jnp.pad(ridx.reshape(_GB, 4, _BR), ((0, 0), (0, 4), (0, 0)))

    p_stacked = r_proj_w.reshape(32, _D, _D).transpose(2, 0, 1).reshape(_D, _D * _D)

    loss = _tc_loss(diff_rows, ridx3, p_stacked, r_emb_w.T)
    return loss[0, 0]


# speedup vs baseline: 1.1417x; 1.1417x over previous
import functools

import jax
import jax.numpy as jnp
from jax import lax
from jax.experimental import pallas as pl
from jax.experimental.pallas import tpu as pltpu
from jax.experimental.pallas import tpu_sc as plsc

_D = 32
_E = 200000
_NW = 32
_CHUNK = 128
_CH = 50
_EPAD = _NW * _CH * _CHUNK
_BT = 2048
_BR = _BT // 4
_GB = _EPAD // _BT


def _sc_gather(node_emb, head_idx, tail_idx):
    mesh = plsc.VectorSubcoreMesh(core_axis_name="c", subcore_axis_name="s")

    @functools.partial(
        pl.kernel,
        mesh=mesh,
        out_type=jax.ShapeDtypeStruct((_EPAD // 4, 128), jnp.float32),
        scratch_types=[
            pltpu.VMEM((_CH, _CHUNK), jnp.int32),
            pltpu.VMEM((_CH, _CHUNK), jnp.int32),
            [pltpu.VMEM((_CHUNK, _D), jnp.float32) for _ in range(2)],
            [pltpu.VMEM((_CHUNK, _D), jnp.float32) for _ in range(2)],
            [pltpu.VMEM((_CHUNK, _D), jnp.float32) for _ in range(2)],
            [pltpu.SemaphoreType.DMA for _ in range(2)],
            [pltpu.SemaphoreType.DMA for _ in range(2)],
        ],
        compiler_params=pltpu.CompilerParams(use_tc_tiling_on_sc=False),
    )
    def gather_kernel(node_hbm, hidx_hbm, tidx_hbm, dout_hbm,
                      hidx_v, tidx_v, hbuf, tbuf, dbuf, sem_g, sem_w):
        wid = lax.axis_index("s") * 2 + lax.axis_index("c")
        pltpu.sync_copy(hidx_hbm.at[wid], hidx_v)
        pltpu.sync_copy(tidx_hbm.at[wid], tidx_v)

        def start_gather(c, b):
            pltpu.make_async_copy(node_hbm.at[hidx_v.at[c]], hbuf[b],
                                  sem_g[b]).start()
            pltpu.make_async_copy(node_hbm.at[tidx_v.at[c]], tbuf[b],
                                  sem_g[b]).start()

        def wait_gather(b):
            pltpu.make_async_copy(node_hbm.at[hidx_v.at[0]], hbuf[b],
                                  sem_g[b]).wait()
            pltpu.make_async_copy(node_hbm.at[tidx_v.at[0]], tbuf[b],
                                  sem_g[b]).wait()

        def wait_write(b):
            pltpu.make_async_copy(
                dbuf[b], dout_hbm.at[pl.ds(0, _CHUNK), pl.ds(0, _D)],
                sem_w[b]).wait()

        def start_write(c, b):
            t = wid * _CH + c
            r0 = (t >> 4) * 512 + (t & 3) * _CHUNK
            lane = ((t >> 2) & 3) * _D
            pltpu.make_async_copy(
                dbuf[b], dout_hbm.at[pl.ds(r0, _CHUNK), pl.ds(lane, _D)],
                sem_w[b]).start()

        start_gather(0, 0)

        def pair(g, carry):
            c0 = g * 2

            @pl.when(g > 0)
            def _w0():
                wait_write(0)

            wait_gather(0)
            start_gather(c0 + 1, 1)

            def sub_row0(r, carry2):
                r4 = r * 4
                for q in range(8):
                    v = (hbuf[0][r4 + (q >> 1), pl.ds((q & 1) * 16, 16)]
                         - tbuf[0][r4 + (q >> 1), pl.ds((q & 1) * 16, 16)])
                    dbuf[0][r4 + (q >> 1), pl.ds((q & 1) * 16, 16)] = v
                return carry2

            lax.fori_loop(0, _CHUNK // 4, sub_row0, 0)
            start_write(c0, 0)

            @pl.when(g > 0)
            def _w1():
                wait_write(1)

            wait_gather(1)

            @pl.when(g < _CH // 2 - 1)
            def _ng():
                start_gather(c0 + 2, 0)

            def sub_row1(r, carry2):
                r4 = r * 4
                for q in range(8):
                    v = (hbuf[1][r4 + (q >> 1), pl.ds((q & 1) * 16, 16)]
                         - tbuf[1][r4 + (q >> 1), pl.ds((q & 1) * 16, 16)])
                    dbuf[1][r4 + (q >> 1), pl.ds((q & 1) * 16, 16)] = v
                return carry2

            lax.fori_loop(0, _CHUNK // 4, sub_row1, 0)
            start_write(c0 + 1, 1)
            return carry

        lax.fori_loop(0, _CH // 2, pair, 0)
        wait_write(0)
        wait_write(1)

    return gather_kernel(node_emb, head_idx, tail_idx)


def _tc_loss(diff2d, ridx3, p_stacked, r_emb_w):
    def body(d_ref, r_ref, p_ref, e_ref, o_ref):
        i = pl.program_id(0)
        blk_t = d_ref[...].T

        growt = lax.shift_right_logical(
            lax.broadcasted_iota(jnp.int32, (_D * _D, _BR), 0), 5)
        rrow = lax.broadcasted_iota(jnp.int32, (64, _BR), 0)
        kcol = lax.broadcasted_iota(jnp.int32, (1, _BR), 1)
        p_t = p_ref[...].astype(jnp.bfloat16)
        e_t = e_ref[...]

        part = jnp.zeros((), jnp.float32)
        for j in range(4):
            d_t = blk_t[_D * j:_D * (j + 1), :].astype(jnp.bfloat16)
            rj = r_ref[0, j, :]
            g = lax.shift_right_logical(rj, 1)

            tile_t = jnp.concatenate([d_t] * _D, axis=0)
            sel = (growt == g[None, :])
            x_t = jnp.where(sel, tile_t, jnp.bfloat16(0))
            out_t = jnp.dot(p_t, x_t,
                            preferred_element_type=jnp.float32)

            onehot_t = (rrow == rj[None, :]).astype(jnp.float32)
            re_t = jnp.dot(e_t, onehot_t,
                           preferred_element_type=jnp.float32)

            s = out_t + re_t
            e_glob = i * _BT + j * _BR + kcol
            s = jnp.where(e_glob < _E, s, 0.0)
            part = part + jnp.sum(s * s)

        @pl.when(i == 0)
        def _init():
            o_ref[...] = jnp.zeros((1, 1), jnp.float32)

        o_ref[...] = o_ref[...] + part

        @pl.when(i == _GB - 1)
        def _final():
            o_ref[...] = o_ref[...] * (1.0 / (_E * _D))

    return pl.pallas_call(
        body,
        grid=(_GB,),
        in_specs=[
            pl.BlockSpec((_BR, 128), lambda i: (i, 0)),
            pl.BlockSpec((1, 8, _BR), lambda i: (i, 0, 0)),
            pl.BlockSpec((_D, _D * _D), lambda i: (0, 0)),
            pl.BlockSpec((_D, 64), lambda i: (0, 0)),
        ],
        out_specs=pl.BlockSpec((1, 1), lambda i: (0, 0)),
        out_shape=jax.ShapeDtypeStruct((1, 1), jnp.float32),
    )(diff2d, ridx3, p_stacked, r_emb_w)


def kernel(node_emb, r_emb_w, r_proj_w, edge_index_t, edge_attr):
    pad = _EPAD - _E
    head_idx = jnp.concatenate(
        [edge_index_t[:, 0], jnp.zeros((pad,), jnp.int32)]).astype(jnp.int32)
    tail_idx = jnp.concatenate(
        [edge_index_t[:, 1], jnp.zeros((pad,), jnp.int32)]).astype(jnp.int32)
    head_idx = head_idx.reshape(_NW, _CH, _CHUNK)
    tail_idx = tail_idx.reshape(_NW, _CH, _CHUNK)

    diff_rows = _sc_gather(node_emb, head_idx, tail_idx)

    ridx = jnp.concatenate(
        [edge_attr[:, 0], jnp.zeros((pad,), jnp.int32)]).astype(jnp.int32)
    ridx3 = jnp.pad(ridx.reshape(_GB, 4, _BR), ((0, 0), (0, 4), (0, 0)))

    p_stacked = r_proj_w.reshape(32, _D, _D).transpose(2, 0, 1).reshape(_D, _D * _D)

    loss = _tc_loss(diff_rows, ridx3, p_stacked, r_emb_w.T)
    return loss[0, 0]
